# baseline (speedup 1.0000x reference)
import jax
import jax.numpy as jnp
from jax import lax
from jax.experimental import pallas as pl
from jax.experimental.pallas import tpu as pltpu
from jax.experimental.pallas import tpu_sc as plsc

_B, _K, _T = 16, 8, 4096
_S = _T + _K
_R = _B * _K
_ROWS_PER_W = _R // 32
_SPECIAL = 2048.0


def _sc_body(p_hbm, out_hbm, data_v, row_v):
    cid = lax.axis_index("c")
    sid = lax.axis_index("s")
    wid = sid * 2 + cid
    iota = lax.iota(jnp.int32, 16)
    for j in range(_ROWS_PER_W):
        r = wid * _ROWS_PER_W + j
        k = lax.rem(r, _K)
        shift = 1 + k
        pltpu.sync_copy(p_hbm.at[r], data_v)
        src0 = iota - shift
        head = plsc.load_gather(data_v, [jnp.maximum(src0, 0)])
        row_v[pl.ds(0, 16)] = jnp.where(src0 >= 0, head, _SPECIAL)

        def mid(i, idx):
            row_v[pl.ds(i * 16, 16)] = plsc.load_gather(data_v, [idx])
            return idx + 16

        lax.fori_loop(1, _T // 16, mid, iota + (16 - shift))
        srct = iota + (_S - 16 - shift)
        tail = plsc.load_gather(data_v, [jnp.minimum(srct, _T - 1)])
        row_v[pl.ds(_S - 16, 16)] = jnp.where(srct < _T, tail, _SPECIAL)
        pltpu.sync_copy(row_v, out_hbm.at[r])


def _mask_body(valid_ref):
    s = lax.broadcasted_iota(jnp.int32, (_K, _S), 1)
    kk = lax.broadcasted_iota(jnp.int32, (_K, _S), 0)
    valid_ref[...] = (s >= 1 + kk) & (s < 1 + kk + _T)


def kernel(prompt):
    p2 = prompt.reshape(_R, _T)
    mesh = plsc.VectorSubcoreMesh(core_axis_name="c", subcore_axis_name="s")
    seq = pl.kernel(
        _sc_body,
        out_type=jax.ShapeDtypeStruct((_R, _S), jnp.float32),
        mesh=mesh,
        scratch_types=[
            pltpu.VMEM((_T,), jnp.float32),
            pltpu.VMEM((_S,), jnp.float32),
        ],
        compiler_params=pltpu.CompilerParams(needs_layout_passes=False),
    )(p2)
    valid = pl.pallas_call(
        _mask_body,
        out_shape=jax.ShapeDtypeStruct((_K, _S), jnp.bool_),
    )()
    return seq.reshape(_B, _K, _S), valid


# speedup vs baseline: 1.0567x; 1.0567x over previous
import jax
import jax.numpy as jnp
from jax import lax
from jax.experimental import pallas as pl
from jax.experimental.pallas import tpu as pltpu
from jax.experimental.pallas import tpu_sc as plsc

_B, _K, _T = 16, 8, 4096
_S = _T + _K
_R = _B * _K
_W = 32
_RPW = _R // _W
_SPECIAL = 2048.0


def _sc_body(p_hbm, out_hbm, data_v, out_v, sem_in, sem_out):
    cid = lax.axis_index("c")
    sid = lax.axis_index("s")
    wid = sid * 2 + cid
    r0 = wid * _RPW
    iota = lax.iota(jnp.int32, 16)
    ins = [
        pltpu.async_copy(
            p_hbm.at[pl.ds((r0 + j) * _T, _T)],
            data_v.at[pl.ds(j * _T, _T)],
            sem_in,
        )
        for j in range(_RPW)
    ]
    outs = []
    for j in range(_RPW):
        ins[j].wait()
        r = r0 + j
        base = (j * _T - 1) + iota - lax.rem(r, _K)
        lo = j * _T
        hi = lo + _T - 1
        head = plsc.load_gather(data_v, [jnp.maximum(base, lo)])
        out_v[pl.ds(j * _S, 16)] = jnp.where(base >= lo, head, _SPECIAL)

        @plsc.parallel_loop(1, _T // 16, unroll=8)
        def _mid(i):
            out_v[pl.ds(j * _S + i * 16, 16)] = plsc.load_gather(
                data_v, [base + i * 16]
            )

        srct = base + (_S - 16)
        tail = plsc.load_gather(data_v, [jnp.minimum(srct, hi)])
        out_v[pl.ds(j * _S + _S - 16, 16)] = jnp.where(srct <= hi, tail, _SPECIAL)
        outs.append(
            pltpu.async_copy(
                out_v.at[pl.ds(j * _S, _S)],
                out_hbm.at[pl.ds(r * _S, _S)],
                sem_out,
            )
        )
    for d in outs:
        d.wait()


def _mask_body(valid_ref):
    s = lax.broadcasted_iota(jnp.int32, (_K, _S), 1)
    kk = lax.broadcasted_iota(jnp.int32, (_K, _S), 0)
    valid_ref[...] = (s >= 1 + kk) & (s < 1 + kk + _T)


def kernel(prompt):
    p1 = prompt.reshape(_R * _T)
    mesh = plsc.VectorSubcoreMesh(core_axis_name="c", subcore_axis_name="s")
    seq = pl.kernel(
        _sc_body,
        out_type=jax.ShapeDtypeStruct((_R * _S,), jnp.float32),
        mesh=mesh,
        scratch_types=[
            pltpu.VMEM((_RPW * _T,), jnp.float32),
            pltpu.VMEM((_RPW * _S,), jnp.float32),
            pltpu.SemaphoreType.DMA,
            pltpu.SemaphoreType.DMA,
        ],
        compiler_params=pltpu.CompilerParams(needs_layout_passes=False),
    )(p1)
    valid = pl.pallas_call(
        _mask_body,
        out_shape=jax.ShapeDtypeStruct((_K, _S), jnp.bool_),
    )()
    return seq.reshape(_B, _K, _S), valid


# speedup vs baseline: 1.2537x; 1.1864x over previous
import jax
import jax.numpy as jnp
from jax import lax
from jax.experimental import pallas as pl
from jax.experimental.pallas import tpu as pltpu
from jax.experimental.pallas import tpu_sc as plsc

_B, _K, _T = 16, 8, 4096
_S = _T + _K
_R = _B * _K
_W = 32
_RPW = _R // _W
_SPECIAL = 2048.0


def _sc_body(p_hbm, out_hbm, *scratch):
    data_v = scratch[:_RPW]
    out_v = scratch[_RPW:2 * _RPW]
    sem_in, sem_out = scratch[2 * _RPW:]
    cid = lax.axis_index("c")
    sid = lax.axis_index("s")
    wid = sid * 2 + cid
    r0 = wid * _RPW
    iota = lax.iota(jnp.int32, 16)
    ins = [
        pltpu.async_copy(p_hbm.at[r0 + j], data_v[j], sem_in)
        for j in range(_RPW)
    ]
    outs = []
    for j in range(_RPW):
        ins[j].wait()
        r = r0 + j
        dj = data_v[j]
        oj = out_v[j]
        base = iota - 1 - lax.rem(r, _K)
        head = plsc.load_gather(dj, [jnp.maximum(base, 0)])
        oj[pl.ds(0, 16)] = jnp.where(base >= 0, head, _SPECIAL)

        @plsc.parallel_loop(1, _T // 16, unroll=8)
        def _mid(i):
            oj[pl.ds(i * 16, 16)] = plsc.load_gather(dj, [base + i * 16])

        srct = base + (_S - 16)
        tail = plsc.load_gather(dj, [jnp.minimum(srct, _T - 1)])
        oj[pl.ds(_S - 16, 16)] = jnp.where(srct < _T, tail, _SPECIAL)
        outs.append(pltpu.async_copy(oj, out_hbm.at[r], sem_out))
    for d in outs:
        d.wait()


def _mask_body(valid_ref):
    s = lax.broadcasted_iota(jnp.int32, (_K, _S), 1)
    kk = lax.broadcasted_iota(jnp.int32, (_K, _S), 0)
    valid_ref[...] = (s >= 1 + kk) & (s < 1 + kk + _T)


def kernel(prompt):
    p2 = prompt.reshape(_R, _T)
    mesh = plsc.VectorSubcoreMesh(core_axis_name="c", subcore_axis_name="s")
    seq = pl.kernel(
        _sc_body,
        out_type=jax.ShapeDtypeStruct((_R, _S), jnp.float32),
        mesh=mesh,
        scratch_types=(
            [pltpu.VMEM((_T,), jnp.float32) for _ in range(_RPW)]
            + [pltpu.VMEM((_S,), jnp.float32) for _ in range(_RPW)]
            + [pltpu.SemaphoreType.DMA, pltpu.SemaphoreType.DMA]
        ),
        compiler_params=pltpu.CompilerParams(needs_layout_passes=False),
    )(p2)
    valid = pl.pallas_call(
        _mask_body,
        out_shape=jax.ShapeDtypeStruct((_K, _S), jnp.bool_),
    )()
    return seq.reshape(_B, _K, _S), valid


# speedup vs baseline: 1.2554x; 1.0013x over previous
import jax
import jax.numpy as jnp
from jax import lax
from jax.experimental import pallas as pl
from jax.experimental.pallas import tpu as pltpu
from jax.experimental.pallas import tpu_sc as plsc

_B, _K, _T = 16, 8, 4096
_S = _T + _K
_R = _B * _K
_W = 32
_RPW = _R // _W
_SPECIAL = 2048.0


def _sc_body(p_hbm, out_hbm, *scratch):
    data_v = scratch[:_RPW]
    out_v = scratch[_RPW:2 * _RPW]
    sem_in, sem_out = scratch[2 * _RPW:]
    cid = lax.axis_index("c")
    sid = lax.axis_index("s")
    wid = sid * 2 + cid
    r0 = wid * _RPW
    iota = lax.iota(jnp.int32, 16)
    ins = [
        pltpu.async_copy(p_hbm.at[r0 + j], data_v[j], sem_in)
        for j in range(_RPW)
    ]
    outs = []
    for j in range(_RPW):
        ins[j].wait()
        r = r0 + j
        dj = data_v[j]
        oj = out_v[j]
        base = iota - 1 - lax.rem(r, _K)
        head = plsc.load_gather(dj, [jnp.maximum(base, 0)])
        oj[pl.ds(0, 16)] = jnp.where(base >= 0, head, _SPECIAL)

        @plsc.parallel_loop(1, _T // 16, unroll=16)
        def _mid(i):
            oj[pl.ds(i * 16, 16)] = plsc.load_gather(dj, [base + i * 16])

        srct = base + (_S - 16)
        tail = plsc.load_gather(dj, [jnp.minimum(srct, _T - 1)])
        oj[pl.ds(_S - 16, 16)] = jnp.where(srct < _T, tail, _SPECIAL)
        outs.append(pltpu.async_copy(oj, out_hbm.at[r], sem_out))
    for d in outs:
        d.wait()


def _mask_body(valid_ref):
    s = lax.broadcasted_iota(jnp.int32, (_K, _S), 1)
    kk = lax.broadcasted_iota(jnp.int32, (_K, _S), 0)
    valid_ref[...] = (s >= 1 + kk) & (s < 1 + kk + _T)


def kernel(prompt):
    p2 = prompt.reshape(_R, _T)
    mesh = plsc.VectorSubcoreMesh(core_axis_name="c", subcore_axis_name="s")
    seq = pl.kernel(
        _sc_body,
        out_type=jax.ShapeDtypeStruct((_R, _S), jnp.float32),
        mesh=mesh,
        scratch_types=(
            [pltpu.VMEM((_T,), jnp.float32) for _ in range(_RPW)]
            + [pltpu.VMEM((_S,), jnp.float32) for _ in range(_RPW)]
            + [pltpu.SemaphoreType.DMA, pltpu.SemaphoreType.DMA]
        ),
        compiler_params=pltpu.CompilerParams(needs_layout_passes=False),
    )(p2)
    valid = pl.pallas_call(
        _mask_body,
        out_shape=jax.ShapeDtypeStruct((_K, _S), jnp.bool_),
    )()
    return seq.reshape(_B, _K, _S), valid


# speedup vs baseline: 1.2621x; 1.0054x over previous
import jax
import jax.numpy as jnp
from jax import lax
from jax.experimental import pallas as pl
from jax.experimental.pallas import tpu as pltpu
from jax.experimental.pallas import tpu_sc as plsc

_B, _K, _T = 16, 8, 4096
_S = _T + _K
_R = _B * _K
_W = 32
_RPW = _R // _W
_SPECIAL = 2048.0


def _sc_body(p_hbm, out_hbm, *scratch):
    data_v = scratch[:_RPW]
    out_v = scratch[_RPW:2 * _RPW]
    sem_in, sem_out = scratch[2 * _RPW:]
    cid = lax.axis_index("c")
    sid = lax.axis_index("s")
    wid = sid * 2 + cid
    r0 = wid * _RPW
    iota = lax.iota(jnp.int32, 16)
    ins = [
        pltpu.async_copy(p_hbm.at[r0 + j], data_v[j], sem_in)
        for j in range(_RPW)
    ]
    outs = []
    for j in range(_RPW):
        ins[j].wait()
        r = r0 + j
        dj = data_v[j]
        oj = out_v[j]
        base = iota - 1 - lax.rem(r, _K)
        head = plsc.load_gather(dj, [jnp.maximum(base, 0)])
        oj[pl.ds(0, 16)] = jnp.where(base >= 0, head, _SPECIAL)

        @plsc.parallel_loop(1, _T // 16, unroll=8)
        def _mid(i):
            oj[pl.ds(i * 16, 16)] = plsc.load_gather(dj, [base + i * 16])

        srct = base + (_S - 16)
        tail = plsc.load_gather(dj, [jnp.minimum(srct, _T - 1)])
        oj[pl.ds(_S - 16, 16)] = jnp.where(srct < _T, tail, _SPECIAL)
        outs.append(pltpu.async_copy(oj, out_hbm.at[r], sem_out))
    for d in outs:
        d.wait()


def _mask_body(valid_ref):
    s = lax.broadcasted_iota(jnp.int32, (_K, _S), 1)
    kk = lax.broadcasted_iota(jnp.int32, (_K, _S), 0)
    valid_ref[...] = (s >= 1 + kk) & (s < 1 + kk + _T)


def kernel(prompt):
    p2 = prompt.reshape(_R, _T)
    mesh = plsc.VectorSubcoreMesh(core_axis_name="c", subcore_axis_name="s")
    seq = pl.kernel(
        _sc_body,
        out_type=jax.ShapeDtypeStruct((_R, _S), jnp.float32),
        mesh=mesh,
        scratch_types=(
            [pltpu.VMEM((_T,), jnp.float32) for _ in range(_RPW)]
            + [pltpu.VMEM((_S,), jnp.float32) for _ in range(_RPW)]
            + [pltpu.SemaphoreType.DMA, pltpu.SemaphoreType.DMA]
        ),
        compiler_params=pltpu.CompilerParams(
            needs_layout_passes=False, skip_device_barrier=True
        ),
    )(p2)
    valid = pl.pallas_call(
        _mask_body,
        out_shape=jax.ShapeDtypeStruct((_K, _S), jnp.bool_),
    )()
    return seq.reshape(_B, _K, _S), valid
